# baseline (speedup 1.0000x reference)
import functools

import jax
import jax.numpy as jnp
from jax import lax
from jax.experimental import pallas as pl
from jax.experimental.pallas import tpu as pltpu
from jax.experimental.pallas import tpu_sc as plsc

HIDDEN = 768
PAD = 1
EPS = 1e-12
L = 16
HV = HIDDEN // L


def _sc_dims():
    try:
        info = plsc.get_sparse_core_info()
        return info.num_cores, info.num_subcores
    except Exception:
        return 2, 16


def _worker_id(num_cores):
    return lax.axis_index("s") * num_cores + lax.axis_index("c")


def _lane_sum(v):
    tot = v[0]
    for j in range(1, L):
        tot = tot + v[j]
    return tot


def _tree_sum(red, v, o):
    red[pl.ds(o, L)] = v
    t = v + red[pl.ds(o + 8, L)]
    red[pl.ds(o, L)] = t
    t = t + red[pl.ds(o + 4, L)]
    red[pl.ds(o, L)] = t
    t = t + red[pl.ds(o + 2, L)]
    red[pl.ds(o, L)] = t
    t = t + red[pl.ds(o + 1, L)]
    return t[0]


def _rsqrt_scalar(x):
    iw = lax.bitcast_convert_type(x, jnp.int32)
    iw = jnp.int32(0x5F3759DF) - lax.shift_right_logical(iw, 1)
    y = lax.bitcast_convert_type(iw, jnp.float32)
    for _ in range(3):
        y = y * (jnp.float32(1.5) - jnp.float32(0.5) * x * y * y)
    return y


def kernel(input_ids, word_emb, pos_emb, type_emb, gamma, beta):
    B, S = input_ids.shape
    N = B * S
    NC, NS = _sc_dims()
    NW = NC * NS
    TPW = N // NW
    CH = 32
    NCH = TPW // CH
    TB = 16
    WPR = S // TPW

    ids_flat = input_ids.reshape(N).astype(jnp.int32)
    pos_fused = pos_emb + type_emb[0][None, :]

    mesh = plsc.VectorSubcoreMesh(
        core_axis_name="c", subcore_axis_name="s",
        num_cores=NC, num_subcores=NS)

    @functools.partial(
        pl.kernel,
        out_type=jax.ShapeDtypeStruct((N, HIDDEN), jnp.float32),
        mesh=mesh,
        scratch_types=[
            pltpu.VMEM((S,), jnp.int32),
            pltpu.VMEM((TPW,), jnp.int32),
            pltpu.VMEM((CH,), jnp.int32),
            pltpu.VMEM((CH,), jnp.int32),
            pltpu.VMEM((CH,), jnp.int32),
            pltpu.VMEM((CH,), jnp.int32),
            pltpu.VMEM((CH, HIDDEN), jnp.float32),
            pltpu.VMEM((CH, HIDDEN), jnp.float32),
            pltpu.VMEM((CH, HIDDEN), jnp.float32),
            pltpu.VMEM((CH, HIDDEN), jnp.float32),
            pltpu.VMEM((HIDDEN,), jnp.float32),
            pltpu.VMEM((HIDDEN,), jnp.float32),
            pltpu.VMEM((24,), jnp.float32),
            pltpu.VMEM((24,), jnp.float32),
            pltpu.SemaphoreType.DMA,
            pltpu.SemaphoreType.DMA,
            pltpu.SemaphoreType.DMA,
            pltpu.SemaphoreType.DMA,
        ],
    )
    def _k(ids_hbm, word_hbm, pos_hbm, gamma_hbm, beta_hbm, out_hbm,
           row_ids, pos_idx, widx0, pidx0, widx1, pidx1,
           wrows0, prows0, wrows1, prows1, gam, bet, red_a, red_b,
           sem_g0, sem_g1, sem_o0, sem_o1):
        wid = _worker_id(NC)
        row = wid // WPR
        off = (wid % WPR) * TPW
        base = row * S + off

        pltpu.sync_copy(ids_hbm.at[pl.ds(row * S, S)], row_ids)
        pltpu.sync_copy(gamma_hbm, gam)
        pltpu.sync_copy(beta_hbm, bet)

        one = jnp.int32(1)
        zero = jnp.int32(0)
        lane = lax.iota(jnp.int32, L)

        def cbody(j, acc_v):
            v = row_ids[pl.ds(j * L, L)]
            return acc_v + jnp.where(v != PAD, one, zero)

        acc_v = lax.fori_loop(0, off // L, cbody, jnp.zeros((L,), jnp.int32))
        carry0 = _lane_sum(acc_v)

        def pbody(i, carry):
            v = row_ids[pl.ds(off + i * L, L)]
            m = jnp.where(v != PAD, one, zero)
            cs = jnp.zeros((L,), jnp.int32)
            acc = carry
            for j in range(L):
                acc = acc + m[j]
                cs = jnp.where(lane == j, acc, cs)
            pos_idx[pl.ds(i * L, L)] = cs * m + PAD
            return acc

        lax.fori_loop(0, TPW // L, pbody, carry0)

        inv_h = jnp.float32(1.0 / HIDDEN)

        def fire_gather(c, widx, pidx, wrows, prows, sem_g):
            for i in range(CH // L):
                widx[pl.ds(i * L, L)] = row_ids[pl.ds(off + c * CH + i * L, L)]
                pidx[pl.ds(i * L, L)] = pos_idx[pl.ds(c * CH + i * L, L)]
            pltpu.async_copy(word_hbm.at[widx], wrows, sem_g)
            pltpu.async_copy(pos_hbm.at[pidx], prows, sem_g)

        def wait_gather(widx, pidx, wrows, prows, sem_g):
            pltpu.make_async_copy(word_hbm.at[widx], wrows, sem_g).wait()
            pltpu.make_async_copy(pos_hbm.at[pidx], prows, sem_g).wait()

        def compute(wrows, prows):
            def tbody(t, _):
                ss = [jnp.zeros((L,), jnp.float32) for _ in range(4)]
                qq = [jnp.zeros((L,), jnp.float32) for _ in range(4)]
                for j in range(HV):
                    sl = pl.ds(j * L, L)
                    v = wrows[t, sl] + prows[t, sl]
                    wrows[t, sl] = v
                    k = j & 3
                    ss[k] = ss[k] + v
                    qq[k] = qq[k] + v * v
                s = (ss[0] + ss[1]) + (ss[2] + ss[3])
                s2 = (qq[0] + qq[1]) + (qq[2] + qq[3])
                mu = _tree_sum(red_a, s, 0) * inv_h
                var = _tree_sum(red_b, s2, 0) * inv_h - mu * mu
                r = _rsqrt_scalar(var + jnp.float32(EPS))
                mur = mu * r
                for j in range(HV):
                    sl = pl.ds(j * L, L)
                    wrows[t, sl] = wrows[t, sl] * r - mur
                return 0

            lax.fori_loop(0, CH, tbody, 0)

            def bbody(sb, _):
                t0 = sb * TB
                for j in range(HV):
                    sl = pl.ds(j * L, L)
                    gj = gam[sl]
                    bj = bet[sl]
                    for tt in range(TB):
                        wrows[t0 + tt, sl] = wrows[t0 + tt, sl] * gj + bj
                return 0

            lax.fori_loop(0, CH // TB, bbody, 0)

        def start_out(c, wrows, sem_o):
            pltpu.async_copy(wrows, out_hbm.at[pl.ds(base + c * CH, CH)], sem_o)

        def wait_out(c, wrows, sem_o):
            pltpu.make_async_copy(
                wrows, out_hbm.at[pl.ds(base + c * CH, CH)], sem_o).wait()

        fire_gather(0, widx0, pidx0, wrows0, prows0, sem_g0)

        def pair_body(c2, _):
            cc = 2 * c2
            wait_gather(widx0, pidx0, wrows0, prows0, sem_g0)

            @pl.when(c2 > 0)
            def _():
                wait_out(cc - 1, wrows1, sem_o1)

            fire_gather(cc + 1, widx1, pidx1, wrows1, prows1, sem_g1)
            compute(wrows0, prows0)
            start_out(cc, wrows0, sem_o0)
            wait_gather(widx1, pidx1, wrows1, prows1, sem_g1)

            @pl.when(cc + 2 < NCH)
            def _():
                wait_out(cc, wrows0, sem_o0)
                fire_gather(cc + 2, widx0, pidx0, wrows0, prows0, sem_g0)

            compute(wrows1, prows1)
            start_out(cc + 1, wrows1, sem_o1)
            return 0

        lax.fori_loop(0, NCH // 2, pair_body, 0)
        wait_out(NCH - 2, wrows0, sem_o0)
        wait_out(NCH - 1, wrows1, sem_o1)

    out = _k(ids_flat, word_emb, pos_fused, gamma, beta)
    return out.reshape(B, S, HIDDEN)


# speedup vs baseline: 1.1362x; 1.1362x over previous
import functools

import jax
import jax.numpy as jnp
from jax import lax
from jax.experimental import pallas as pl
from jax.experimental.pallas import tpu as pltpu
from jax.experimental.pallas import tpu_sc as plsc

HIDDEN = 768
PAD = 1
EPS = 1e-12
L = 16
HV = HIDDEN // L


def _sc_dims():
    try:
        info = plsc.get_sparse_core_info()
        return info.num_cores, info.num_subcores
    except Exception:
        return 2, 16


def _worker_id(num_cores):
    return lax.axis_index("s") * num_cores + lax.axis_index("c")


def _lane_sum(v):
    tot = v[0]
    for j in range(1, L):
        tot = tot + v[j]
    return tot


def _tree_sum(red, v, o):
    red[pl.ds(o, L)] = v
    t = v + red[pl.ds(o + 8, L)]
    red[pl.ds(o, L)] = t
    t = t + red[pl.ds(o + 4, L)]
    red[pl.ds(o, L)] = t
    t = t + red[pl.ds(o + 2, L)]
    red[pl.ds(o, L)] = t
    t = t + red[pl.ds(o + 1, L)]
    return t[0]


def _rsqrt_scalar(x):
    iw = lax.bitcast_convert_type(x, jnp.int32)
    iw = jnp.int32(0x5F3759DF) - lax.shift_right_logical(iw, 1)
    y = lax.bitcast_convert_type(iw, jnp.float32)
    for _ in range(3):
        y = y * (jnp.float32(1.5) - jnp.float32(0.5) * x * y * y)
    return y


def kernel(input_ids, word_emb, pos_emb, type_emb, gamma, beta):
    B, S = input_ids.shape
    N = B * S
    NC, NS = _sc_dims()
    NW = NC * NS
    TPW = N // NW
    CH = 32
    NCH = TPW // CH
    TB = 16
    WPR = S // TPW

    ids_flat = input_ids.reshape(N).astype(jnp.int32)
    pos_fused = pos_emb + type_emb[0][None, :]

    mesh = plsc.VectorSubcoreMesh(
        core_axis_name="c", subcore_axis_name="s",
        num_cores=NC, num_subcores=NS)

    @functools.partial(
        pl.kernel,
        out_type=jax.ShapeDtypeStruct((N, HIDDEN), jnp.float32),
        mesh=mesh,
        scratch_types=[
            pltpu.VMEM((S,), jnp.int32),
            pltpu.VMEM((TPW,), jnp.int32),
            pltpu.VMEM((CH,), jnp.int32),
            pltpu.VMEM((CH,), jnp.int32),
            pltpu.VMEM((CH,), jnp.int32),
            pltpu.VMEM((CH,), jnp.int32),
            pltpu.VMEM((CH, HIDDEN), jnp.float32),
            pltpu.VMEM((CH, HIDDEN), jnp.float32),
            pltpu.VMEM((CH, HIDDEN), jnp.float32),
            pltpu.VMEM((CH, HIDDEN), jnp.float32),
            pltpu.VMEM((HIDDEN,), jnp.float32),
            pltpu.VMEM((HIDDEN,), jnp.float32),
            pltpu.VMEM((24,), jnp.float32),
            pltpu.VMEM((24,), jnp.float32),
            pltpu.VMEM((64,), jnp.float32),
            pltpu.SemaphoreType.DMA,
            pltpu.SemaphoreType.DMA,
            pltpu.SemaphoreType.DMA,
            pltpu.SemaphoreType.DMA,
        ],
    )
    def _k(ids_hbm, word_hbm, pos_hbm, gamma_hbm, beta_hbm, out_hbm,
           row_ids, pos_idx, widx0, pidx0, widx1, pidx1,
           wrows0, prows0, wrows1, prows1, gam, bet, red_a, red_b, rmu,
           sem_g0, sem_g1, sem_o0, sem_o1):
        wid = _worker_id(NC)
        row = wid // WPR
        off = (wid % WPR) * TPW
        base = row * S + off

        pltpu.sync_copy(ids_hbm.at[pl.ds(row * S, S)], row_ids)
        pltpu.sync_copy(gamma_hbm, gam)
        pltpu.sync_copy(beta_hbm, bet)

        one = jnp.int32(1)
        zero = jnp.int32(0)
        lane = lax.iota(jnp.int32, L)

        def cbody(j, acc_v):
            v = row_ids[pl.ds(j * L, L)]
            return acc_v + jnp.where(v != PAD, one, zero)

        acc_v = lax.fori_loop(0, off // L, cbody, jnp.zeros((L,), jnp.int32))
        carry0 = _lane_sum(acc_v)

        def pbody(i, carry):
            v = row_ids[pl.ds(off + i * L, L)]
            m = jnp.where(v != PAD, one, zero)
            cs = jnp.zeros((L,), jnp.int32)
            acc = carry
            for j in range(L):
                acc = acc + m[j]
                cs = jnp.where(lane == j, acc, cs)
            pos_idx[pl.ds(i * L, L)] = cs * m + PAD
            return acc

        lax.fori_loop(0, TPW // L, pbody, carry0)

        inv_h = jnp.float32(1.0 / HIDDEN)

        def fire_gather(c, widx, pidx, wrows, prows, sem_g):
            for i in range(CH // L):
                widx[pl.ds(i * L, L)] = row_ids[pl.ds(off + c * CH + i * L, L)]
                pidx[pl.ds(i * L, L)] = pos_idx[pl.ds(c * CH + i * L, L)]
            pltpu.async_copy(word_hbm.at[widx], wrows, sem_g)
            pltpu.async_copy(pos_hbm.at[pidx], prows, sem_g)

        def wait_gather(widx, pidx, wrows, prows, sem_g):
            pltpu.make_async_copy(word_hbm.at[widx], wrows, sem_g).wait()
            pltpu.make_async_copy(pos_hbm.at[pidx], prows, sem_g).wait()

        def compute(wrows, prows):
            def tbody(t, _):
                ss = [jnp.zeros((L,), jnp.float32) for _ in range(4)]
                qq = [jnp.zeros((L,), jnp.float32) for _ in range(4)]
                for j in range(HV):
                    sl = pl.ds(j * L, L)
                    v = wrows[t, sl] + prows[t, sl]
                    wrows[t, sl] = v
                    k = j & 3
                    ss[k] = ss[k] + v
                    qq[k] = qq[k] + v * v
                s = (ss[0] + ss[1]) + (ss[2] + ss[3])
                s2 = (qq[0] + qq[1]) + (qq[2] + qq[3])
                mu = _tree_sum(red_a, s, 0) * inv_h
                var = _tree_sum(red_b, s2, 0) * inv_h - mu * mu
                r = _rsqrt_scalar(var + jnp.float32(EPS))
                g = t // TB
                tl = t - g * TB
                rv = jnp.where(lane == tl, jnp.full((L,), r, jnp.float32),
                               rmu[pl.ds(g * 2 * L, L)])
                mv = jnp.where(lane == tl, jnp.full((L,), mu * r, jnp.float32),
                               rmu[pl.ds(g * 2 * L + L, L)])
                rmu[pl.ds(g * 2 * L, L)] = rv
                rmu[pl.ds(g * 2 * L + L, L)] = mv
                return 0

            lax.fori_loop(0, CH, tbody, 0)

            def bbody(sb, _):
                t0 = sb * TB
                rv = rmu[pl.ds(sb * 2 * L, L)]
                mv = rmu[pl.ds(sb * 2 * L + L, L)]
                rs = [rv[tt] for tt in range(TB)]
                ms = [mv[tt] for tt in range(TB)]
                for j in range(HV):
                    sl = pl.ds(j * L, L)
                    gj = gam[sl]
                    bj = bet[sl]
                    for tt in range(TB):
                        z = wrows[t0 + tt, sl] * rs[tt] - ms[tt]
                        wrows[t0 + tt, sl] = z * gj + bj
                return 0

            lax.fori_loop(0, CH // TB, bbody, 0)

        def start_out(c, wrows, sem_o):
            pltpu.async_copy(wrows, out_hbm.at[pl.ds(base + c * CH, CH)], sem_o)

        def wait_out(c, wrows, sem_o):
            pltpu.make_async_copy(
                wrows, out_hbm.at[pl.ds(base + c * CH, CH)], sem_o).wait()

        fire_gather(0, widx0, pidx0, wrows0, prows0, sem_g0)

        def pair_body(c2, _):
            cc = 2 * c2
            wait_gather(widx0, pidx0, wrows0, prows0, sem_g0)

            @pl.when(c2 > 0)
            def _():
                wait_out(cc - 1, wrows1, sem_o1)

            fire_gather(cc + 1, widx1, pidx1, wrows1, prows1, sem_g1)
            compute(wrows0, prows0)
            start_out(cc, wrows0, sem_o0)
            wait_gather(widx1, pidx1, wrows1, prows1, sem_g1)

            @pl.when(cc + 2 < NCH)
            def _():
                wait_out(cc, wrows0, sem_o0)
                fire_gather(cc + 2, widx0, pidx0, wrows0, prows0, sem_g0)

            compute(wrows1, prows1)
            start_out(cc + 1, wrows1, sem_o1)
            return 0

        lax.fori_loop(0, NCH // 2, pair_body, 0)
        wait_out(NCH - 2, wrows0, sem_o0)
        wait_out(NCH - 1, wrows1, sem_o1)

    out = _k(ids_flat, word_emb, pos_fused, gamma, beta)
    return out.reshape(B, S, HIDDEN)
